# baseline (speedup 1.0000x reference)
import jax
import jax.numpy as jnp
from jax.experimental import pallas as pl
from jax.experimental.pallas import tpu as pltpu

_K = 8
_H = 16
_T = 2048
_S = 2048
_SAMPLED_HEAD = 12
_SLAB = 8
_BR = 1024
_NB = (_H * _T) // _BR


def _copy_patch_kernel(flat_ref, slab_ref, out_ref):
    i = pl.program_id(0)
    out_ref[...] = flat_ref[...]

    @pl.when((i % 2) == 1)
    def _():
        h = (i - 1) // 2
        x = slab_ref[:, _SLAB - 1, :]
        m = jnp.max(x, axis=-1, keepdims=True)
        idx = jax.lax.broadcasted_iota(jnp.int32, x.shape, 1)
        arg = jnp.min(jnp.where(x == m, idx, _S), axis=-1)
        eq = (arg[:, None] == arg[None, :]).astype(jnp.int32)
        maxcount = jnp.max(jnp.sum(eq, axis=1))
        cond = maxcount <= _K
        own = slab_ref[pl.ds(h, 1), _SLAB - 1, :]
        newrow = jnp.where(cond, x[_SAMPLED_HEAD, :][None, :], own)
        out_ref[pl.ds(_BR - 1, 1), :] = newrow


def kernel(attention_weight):
    flat = attention_weight.reshape(_H * _T, _S)
    slab3 = attention_weight.reshape(_H, _T, _S)
    last_blk = (_T - _SLAB) // _SLAB
    out = pl.pallas_call(
        _copy_patch_kernel,
        grid=(_NB,),
        in_specs=[
            pl.BlockSpec((_BR, _S), lambda i: (i, 0)),
            pl.BlockSpec((_H, _SLAB, _S), lambda i: (0, last_blk, 0)),
        ],
        out_specs=pl.BlockSpec((_BR, _S), lambda i: (i, 0)),
        out_shape=jax.ShapeDtypeStruct((_H * _T, _S), jnp.float32),
        compiler_params=pltpu.CompilerParams(
            dimension_semantics=("parallel",),
        ),
    )(flat, slab3)
    return out.reshape(1, _H, _T, _S)
